# Initial kernel scaffold; baseline (speedup 1.0000x reference)
import jax
import jax.numpy as jnp
from jax.experimental import pallas as pl


def kernel(tool_ids, table):
    raise NotImplementedError("write your pallas kernel here")


# speedup vs baseline: 1.8665x; 1.8665x over previous
import functools

import jax
import jax.numpy as jnp
from jax import lax
from jax.experimental import pallas as pl
from jax.experimental.pallas import tpu as pltpu
from jax.experimental.pallas import tpu_sc as plsc

_HIDDEN = 64
_B = 16384 * 50
_NC, _NS = 2, 16
_NW = _NC * _NS
_BPW = _B // _NW
_CHUNK = 128
_NBUF = 4
_NCHUNK = _BPW // _CHUNK
_NGROUP = _NCHUNK // _NBUF


def _gather_sc(ids2d, table):
  mesh = plsc.VectorSubcoreMesh(core_axis_name="c", subcore_axis_name="s")

  @functools.partial(
      pl.kernel,
      out_type=jax.ShapeDtypeStruct((_B, _HIDDEN), jnp.float32),
      mesh=mesh,
      compiler_params=pltpu.CompilerParams(use_tc_tiling_on_sc=False),
      scratch_types=(
          [pltpu.VMEM((_NCHUNK, _CHUNK), jnp.int32)]
          + [pltpu.VMEM((_CHUNK, _HIDDEN), jnp.float32) for _ in range(_NBUF)]
          + [pltpu.SemaphoreType.DMA for _ in range(2 * _NBUF + 1)]
      ),
  )
  def body(ids_hbm, table_hbm, out_hbm, idx_v, *rest):
    rows = rest[:_NBUF]
    gsem = rest[_NBUF:2 * _NBUF]
    osem = rest[2 * _NBUF:3 * _NBUF]
    isem = rest[3 * _NBUF]
    wid = lax.axis_index("s") * _NC + lax.axis_index("c")
    base = wid * _BPW

    pltpu.async_copy(ids_hbm.at[pl.ds(wid * _NCHUNK, _NCHUNK)], idx_v,
                     isem).wait()

    def fire_gather(b, ci):
      pltpu.async_copy(table_hbm.at[idx_v.at[ci]], rows[b], gsem[b])

    def wait_gather(b, ci):
      pltpu.make_async_copy(table_hbm.at[idx_v.at[ci]], rows[b],
                            gsem[b]).wait()

    for b in range(_NBUF):
      fire_gather(b, b)

    @pl.loop(0, _NGROUP)
    def _group(t):
      c0 = t * _NBUF
      for b in range(_NBUF):
        ci = c0 + b
        wait_gather(b, ci)
        pltpu.async_copy(rows[b],
                         out_hbm.at[pl.ds(base + ci * _CHUNK, _CHUNK)],
                         osem[b])
      c2 = jnp.minimum(t + 1, _NGROUP - 1) * _NBUF
      for b in range(_NBUF):
        pltpu.make_async_copy(rows[b],
                              out_hbm.at[pl.ds(base, _CHUNK)],
                              osem[b]).wait()
        fire_gather(b, c2 + b)

    for b in range(_NBUF):
      wait_gather(b, b)

  return body(ids2d, table)


def kernel(tool_ids, table):
  ids2d = tool_ids.astype(jnp.int32).reshape(_B // _CHUNK, _CHUNK)
  out = _gather_sc(ids2d, table)
  return out.reshape(tool_ids.shape + (table.shape[-1],))
